# baseline (speedup 1.0000x reference)
import functools

import jax
import jax.numpy as jnp
from jax import lax
from jax.experimental import pallas as pl
from jax.experimental.pallas import tpu as pltpu
from jax.experimental.pallas import tpu_sc as plsc

D = 64
B = 16384

NC = 2
NS = 16
NW = NC * NS
BPW = B // NW
CH = 128
NCH = BPW // CH
L = 16


def _glove_body(idx_hbm, w_hbm, bw_hbm, u_hbm, bu_hbm, out_hbm,
                idx_i, idx_j, w_rows, u_rows, bw_v, bu_v, out_v, sem):
    cid = lax.axis_index("c")
    sid = lax.axis_index("s")
    wid = sid * NC + cid
    base = wid * BPW

    pltpu.sync_copy(idx_hbm.at[0, pl.ds(base, BPW)], idx_i)
    pltpu.sync_copy(idx_hbm.at[1, pl.ds(base, BPW)], idx_j)

    copies = []
    for c in range(NCH):
        ii = idx_i.at[pl.ds(c * CH, CH)]
        jj = idx_j.at[pl.ds(c * CH, CH)]
        sl = pl.ds(c * CH, CH)
        copies.append(pltpu.async_copy(w_hbm.at[ii], w_rows.at[sl], sem))
        copies.append(pltpu.async_copy(u_hbm.at[jj], u_rows.at[sl], sem))
        copies.append(pltpu.async_copy(bw_hbm.at[ii], bw_v.at[sl], sem))
        copies.append(pltpu.async_copy(bu_hbm.at[jj], bu_v.at[sl], sem))
    for cp in copies:
        cp.wait()

    def init_body(g, _):
        sl = pl.ds(g * L, L)
        out_v[sl] = bw_v[sl] + bu_v[sl]
        return 0

    lax.fori_loop(0, BPW // L, init_body, 0)

    def row_body(r, _):
        acc = w_rows[r, pl.ds(0, L)] * u_rows[r, pl.ds(0, L)]
        for q in range(1, D // L):
            acc = acc + w_rows[r, pl.ds(q * L, L)] * u_rows[r, pl.ds(q * L, L)]
        ridx = jnp.full((L,), r, jnp.int32)
        plsc.addupdate_scatter(out_v, [ridx], acc)
        return 0

    lax.fori_loop(0, BPW, row_body, 0)

    pltpu.sync_copy(out_v, out_hbm.at[pl.ds(base, BPW)])


@jax.jit
def _glove(indices, W, b_w, U, b_u):
    mesh = plsc.VectorSubcoreMesh(core_axis_name="c", subcore_axis_name="s")
    fn = pl.kernel(
        _glove_body,
        mesh=mesh,
        compiler_params=pltpu.CompilerParams(
            needs_layout_passes=False, use_tc_tiling_on_sc=False),
        out_type=jax.ShapeDtypeStruct((B,), jnp.float32),
        scratch_types=[
            pltpu.VMEM((BPW,), jnp.int32),
            pltpu.VMEM((BPW,), jnp.int32),
            pltpu.VMEM((BPW, D), jnp.float32),
            pltpu.VMEM((BPW, D), jnp.float32),
            pltpu.VMEM((BPW,), jnp.float32),
            pltpu.VMEM((BPW,), jnp.float32),
            pltpu.VMEM((BPW,), jnp.float32),
            pltpu.SemaphoreType.DMA,
        ],
    )
    return fn(indices, W, b_w, U, b_u)


def kernel(indices, W, b_w, U, b_u):
    return _glove(indices.astype(jnp.int32), W, b_w, U, b_u)
